# baseline (speedup 1.0000x reference)
import functools

import jax
import jax.numpy as jnp
import numpy as np
from jax.experimental import pallas as pl
from jax.experimental.pallas import tpu as pltpu

_T = 2.0
_LOG_2PI = float(np.log(2.0 * np.pi))
_N = 16384
_D = 64
_C = 128
_BLOCK = 2048


def _body(theta_ref, ctx_ref, w_ref, b_ref, out_ref):
    ctx = ctx_ref[...]
    acts = jnp.dot(ctx, w_ref[...], preferred_element_type=jnp.float32)
    acts = acts + b_ref[...]
    mu = acts[:, :_D]
    log_sigma = jnp.tanh(acts[:, _D:])
    z = (theta_ref[...] - mu) * jnp.exp(-log_sigma)
    row = jnp.sum(z * z + 2.0 * log_sigma, axis=-1)
    out_ref[...] = (-0.5 / _T) * row + (-0.5 * _D * _LOG_2PI / _T)


@functools.partial(jax.jit, static_argnames=())
def kernel(theta, context, W_mu, b_mu, W_ls, b_ls):
    w = jnp.concatenate([W_mu, W_ls], axis=1)
    b = jnp.concatenate([b_mu, b_ls])[None, :]
    n = theta.shape[0]
    grid = (n // _BLOCK,)
    return pl.pallas_call(
        _body,
        grid=grid,
        in_specs=[
            pl.BlockSpec((_BLOCK, _D), lambda i: (i, 0)),
            pl.BlockSpec((_BLOCK, _C), lambda i: (i, 0)),
            pl.BlockSpec((_C, 2 * _D), lambda i: (0, 0)),
            pl.BlockSpec((1, 2 * _D), lambda i: (0, 0)),
        ],
        out_specs=pl.BlockSpec((_BLOCK,), lambda i: (i,)),
        out_shape=jax.ShapeDtypeStruct((n,), jnp.float32),
        compiler_params=pltpu.CompilerParams(
            dimension_semantics=("parallel",),
        ),
    )(theta, context, w, b)


# speedup vs baseline: 1.3084x; 1.3084x over previous
import functools

import jax
import jax.numpy as jnp
import numpy as np
from jax import lax
from jax.experimental import pallas as pl
from jax.experimental.pallas import tpu as pltpu

_T = 2.0
_LOG_2PI = float(np.log(2.0 * np.pi))
_D = 64
_C = 128
_BLOCK = 2048
_DN = (((0,), (1,)), ((), ()))


def _body(theta_ref, ctx_ref, w_ref, b_ref, eye_ref, out_ref):
    actsT = lax.dot_general(w_ref[...], ctx_ref[...], _DN,
                            preferred_element_type=jnp.float32)
    thetaT = lax.dot_general(eye_ref[...], theta_ref[...], _DN,
                             preferred_element_type=jnp.float32)
    b = b_ref[...]
    mu = actsT[:_D] + b[:_D]
    ls = jnp.tanh(actsT[_D:] + b[_D:])
    z = (thetaT - mu) * jnp.exp(-ls)
    vals = z * z + 2.0 * ls
    out_ref[...] = (-0.5 / _T) * jnp.sum(vals, axis=0) + (-0.5 * _D * _LOG_2PI / _T)


@functools.partial(jax.jit, static_argnames=())
def kernel(theta, context, W_mu, b_mu, W_ls, b_ls):
    w = jnp.concatenate([W_mu, W_ls], axis=1)
    b = jnp.concatenate([b_mu, b_ls])[:, None]
    eye = jnp.eye(_D, dtype=jnp.float32)
    n = theta.shape[0]
    grid = (n // _BLOCK,)
    return pl.pallas_call(
        _body,
        grid=grid,
        in_specs=[
            pl.BlockSpec((_BLOCK, _D), lambda i: (i, 0)),
            pl.BlockSpec((_BLOCK, _C), lambda i: (i, 0)),
            pl.BlockSpec((_C, 2 * _D), lambda i: (0, 0)),
            pl.BlockSpec((2 * _D, 1), lambda i: (0, 0)),
            pl.BlockSpec((_D, _D), lambda i: (0, 0)),
        ],
        out_specs=pl.BlockSpec((_BLOCK,), lambda i: (i,)),
        out_shape=jax.ShapeDtypeStruct((n,), jnp.float32),
        compiler_params=pltpu.CompilerParams(
            dimension_semantics=("parallel",),
        ),
    )(theta, context, w, b, eye)
